# baseline (speedup 1.0000x reference)
import functools

import jax
import jax.numpy as jnp
from jax import lax
from jax.experimental import pallas as pl
from jax.experimental.pallas import tpu as pltpu
from jax.experimental.pallas import tpu_sc as plsc

L = 16
C = 8
D1 = 128
D2 = 128
D3 = 512
N = 8388608


@functools.partial(
    pl.kernel,
    out_type=jax.ShapeDtypeStruct((L,), jnp.int32),
    mesh=plsc.ScalarSubcoreMesh(axis_name="c", num_cores=1),
    compiler_params=pltpu.CompilerParams(use_tc_tiling_on_sc=False),
    scratch_types=[
        pltpu.SMEM((L + D1,), jnp.float32),
        pltpu.SMEM((D2, C), jnp.float32),
        pltpu.SMEM((1, D3), jnp.float32),
        pltpu.SMEM((L,), jnp.int32),
        pltpu.SemaphoreType.DMA,
    ],
)
def _search(params_hbm, bounds_hbm, out_hbm,
            par_s, s2_s, s3_s, out_s, sem):
    pltpu.async_copy(params_hbm, par_s, sem).wait()
    xs = par_s[0]

    def bsearch(load, size):
        pos = jnp.int32(0)
        w = size // 2
        while w >= 1:
            t = load(pos + (w - 1))
            pos = pos + jnp.where(t < xs, w, 0)
            w //= 2
        return pos

    i1 = jnp.maximum(bsearch(lambda i: par_s[L + i], D1) - 1, 0)
    copies = [
        pltpu.async_copy(
            bounds_hbm.at[pl.ds(i1 * D2 + j, 1), pl.ds(0, C)],
            s2_s.at[pl.ds(j, 1), pl.ds(0, C)], sem)
        for j in range(D2)
    ]
    for cp in copies:
        cp.wait()
    j1 = jnp.maximum(bsearch(lambda i: s2_s[i, 0], D2) - 1, 0)
    row = i1 * D2 + j1
    pltpu.async_copy(
        bounds_hbm.at[pl.ds(row, 1), pl.ds(0, D3)], s3_s, sem).wait()
    idx = row * D3 + bsearch(lambda i: s3_s[0, i], D3)

    for i in range(L):
        out_s[i] = idx
    pltpu.sync_copy(out_s, out_hbm)


def kernel(x, boundaries, out_int32, right, out):
    xq = jnp.asarray(x, dtype=boundaries.dtype)
    xadj = jnp.where(jnp.asarray(right, jnp.int32) != 0,
                     jnp.nextafter(xq, jnp.inf), xq)
    params = jnp.concatenate(
        [jnp.full((L,), xadj, dtype=jnp.float32),
         boundaries[::D2 * D3]])
    res = _search(params, boundaries.reshape(D1 * D2, D3))
    return res[0].astype(jnp.int32)


# speedup vs baseline: 3.4262x; 3.4262x over previous
import functools

import jax
import jax.numpy as jnp
from jax import lax
from jax.experimental import pallas as pl
from jax.experimental.pallas import tpu as pltpu
from jax.experimental.pallas import tpu_sc as plsc

L = 16
K = 128
K2 = 64
S1 = 65536
S2 = 1024
W3 = 1024
N = 8388608


@functools.partial(
    pl.kernel,
    out_type=jax.ShapeDtypeStruct((L,), jnp.int32),
    mesh=plsc.VectorSubcoreMesh(core_axis_name="c", subcore_axis_name="s",
                                num_cores=1, num_subcores=1),
    scratch_types=[
        pltpu.VMEM((K,), jnp.int32),
        pltpu.VMEM((K,), jnp.float32),
        pltpu.VMEM((W3,), jnp.float32),
        pltpu.VMEM((L,), jnp.float32),
        pltpu.VMEM((L,), jnp.int32),
        pltpu.SemaphoreType.DMA,
        pltpu.SemaphoreType.DMA,
    ],
)
def _search(params_hbm, bounds_hbm, out_hbm,
            idx_v, vals_v, last_v, par_v, out_v, sem, sem2):
    only_tile0 = jnp.logical_and(lax.axis_index("c") == 0,
                                 lax.axis_index("s") == 0)

    @pl.when(only_tile0)
    def _():
        iota = lax.iota(jnp.int32, L)
        ones = jnp.ones((L,), jnp.int32)
        zeros = jnp.zeros((L,), jnp.int32)

        for k in range(K // L):
            idx_v[pl.ds(k * L, L)] = (k * L + iota) * S1
        l1 = pltpu.async_copy(bounds_hbm.at[idx_v], vals_v, sem)
        pc = pltpu.async_copy(params_hbm, par_v, sem2)
        pc.wait()
        xv = par_v[...]
        xs = xv[0]
        l1.wait()

        def bsearch(ref, size):
            pos = jnp.int32(0)
            w = size // 2
            while w >= L:
                v = ref[pl.ds(pos + (w - L), L)]
                pos = pos + jnp.where(v[L - 1] < xs, w, 0)
                w //= 2
            v = ref[pl.ds(pos, L)]
            acc = jnp.where(v < xv, ones, zeros)
            lanes = [acc[j] for j in range(L)]
            while len(lanes) > 1:
                lanes = [lanes[i] + lanes[i + 1]
                         for i in range(0, len(lanes), 2)]
            return pos + lanes[0]

        c1 = bsearch(vals_v, K)
        base = jnp.maximum(c1 - 1, 0) * S1

        bb = jnp.full((L,), base, jnp.int32)
        for k in range(K2 // L):
            idx_v[pl.ds(k * L, L)] = bb + (k * L + iota) * S2
        pltpu.async_copy(bounds_hbm.at[idx_v.at[pl.ds(0, K2)]],
                         vals_v.at[pl.ds(0, K2)], sem).wait()
        c2 = bsearch(vals_v, K2)
        base = base + jnp.maximum(c2 - 1, 0) * S2

        pltpu.async_copy(bounds_hbm.at[pl.ds(base, W3)], last_v, sem).wait()
        idx = base + bsearch(last_v, W3)

        out_v[...] = jnp.full((L,), idx, jnp.int32)
        pltpu.sync_copy(out_v, out_hbm)


def kernel(x, boundaries, out_int32, right, out):
    xq = jnp.asarray(x, dtype=boundaries.dtype)
    xadj = jnp.where(jnp.asarray(right, jnp.int32) != 0,
                     jnp.nextafter(xq, jnp.inf), xq)
    params = jnp.full((L,), xadj, dtype=jnp.float32)
    res = _search(params, boundaries)
    return res[0].astype(jnp.int32)
